# baseline (speedup 1.0000x reference)
import functools

import jax
import jax.numpy as jnp
from jax import lax
from jax.experimental import pallas as pl
from jax.experimental.pallas import tpu as pltpu
from jax.experimental.pallas import tpu_sc as plsc

_T = 2048
_D = 1024
_H = 1024
_N = 1024
_E = 8
_F = 2048
_BT = 256
_LANES = 128
_BE = 256
_CAP = 2048
_NBC = _E * (_CAP // _BE)
_ROWS = _NBC * _BE

_NC = 2
_NS = 16
_NW = _NC * _NS
_TPW = _T // _NW
_CH = 16
_NCHUNK = _TPW // _CH


def _router_body(x_ref, wp_ref, bp_ref, wr_ref, br_ref,
                 h_ref, g1_ref, g2_ref, p1_ref, p2_ref,
                 counts_ref, aux_ref, cnt_acc, imp_acc):
    pid = pl.program_id(0)
    nblk = pl.num_programs(0)

    @pl.when(pid == 0)
    def _init():
        cnt_acc[...] = jnp.zeros_like(cnt_acc)
        imp_acc[...] = jnp.zeros_like(imp_acc)

    x = x_ref[...]
    h = jax.nn.gelu(jnp.dot(x, wp_ref[...], preferred_element_type=jnp.float32)
                    + bp_ref[...])
    h_ref[...] = h

    logits = jnp.dot(h, wr_ref[...], preferred_element_type=jnp.float32) + br_ref[...]
    lmax = jnp.max(logits, axis=-1, keepdims=True)
    ex = jnp.exp(logits - lmax)
    probs = ex / jnp.sum(ex, axis=-1, keepdims=True)

    lane = jax.lax.broadcasted_iota(jnp.int32, probs.shape, 1)
    v1 = jnp.max(probs, axis=-1, keepdims=True)
    i1 = jnp.min(jnp.where(probs >= v1, lane, _LANES), axis=-1, keepdims=True)
    m1 = lane == i1
    probs_m = jnp.where(m1, -1.0, probs)
    v2 = jnp.max(probs_m, axis=-1, keepdims=True)
    i2 = jnp.min(jnp.where(probs_m >= v2, lane, _LANES), axis=-1, keepdims=True)
    m2 = lane == i2

    s = v1 + v2
    g1 = v1 / s
    g2 = v2 / s

    mask = (m1 | m2).astype(jnp.float32)
    row = jax.lax.broadcasted_iota(jnp.int32, (_BT, _BT), 0)
    col = jax.lax.broadcasted_iota(jnp.int32, (_BT, _BT), 1)
    ltri = (col < row).astype(jnp.float32)
    excl = jnp.dot(ltri, mask, preferred_element_type=jnp.float32)
    carry = cnt_acc[...]
    rank = excl + carry
    r1 = jnp.sum(jnp.where(m1, rank, 0.0), axis=-1, keepdims=True)
    r2 = jnp.sum(jnp.where(m2, rank, 0.0), axis=-1, keepdims=True)

    pos1 = i1.astype(jnp.float32) * _CAP + r1
    pos2 = i2.astype(jnp.float32) * _CAP + r2

    ones = jnp.ones((1, _LANES), jnp.float32)
    g1_ref[...] = g1 * ones
    g2_ref[...] = g2 * ones
    p1_ref[...] = (pos1 * ones).astype(jnp.int32)
    p2_ref[...] = (pos2 * ones).astype(jnp.int32)

    cnt_new = carry + jnp.sum(mask, axis=0, keepdims=True)
    imp_new = imp_acc[...] + jnp.sum(probs, axis=0, keepdims=True)
    cnt_acc[...] = cnt_new
    imp_acc[...] = imp_new

    @pl.when(pid == nblk - 1)
    def _fin():
        counts_ref[...] = cnt_new
        aux = _E * jnp.sum((imp_new / _T) * (cnt_new / _T))
        aux_ref[...] = jnp.full((1, _LANES), aux, jnp.float32)


def _router_call(x, W_proj, b_proj2, wr_pad, br_pad):
    nblk = _T // _BT
    out_shapes = (
        jax.ShapeDtypeStruct((_T, _H), jnp.float32),
        jax.ShapeDtypeStruct((_T, _LANES), jnp.float32),
        jax.ShapeDtypeStruct((_T, _LANES), jnp.float32),
        jax.ShapeDtypeStruct((_T, _LANES), jnp.int32),
        jax.ShapeDtypeStruct((_T, _LANES), jnp.int32),
        jax.ShapeDtypeStruct((1, _LANES), jnp.float32),
        jax.ShapeDtypeStruct((1, _LANES), jnp.float32),
    )
    tok_spec = lambda w: pl.BlockSpec((_BT, w), lambda i: (i, 0))
    const_spec = lambda a, b: pl.BlockSpec((a, b), lambda i: (0, 0))
    return pl.pallas_call(
        _router_body,
        grid=(nblk,),
        in_specs=[
            tok_spec(_D),
            const_spec(_D, _H),
            const_spec(1, _H),
            const_spec(_H, _LANES),
            const_spec(1, _LANES),
        ],
        out_specs=(
            tok_spec(_H),
            tok_spec(_LANES), tok_spec(_LANES),
            tok_spec(_LANES), tok_spec(_LANES),
            const_spec(1, _LANES), const_spec(1, _LANES),
        ),
        out_shape=out_shapes,
        scratch_shapes=[
            pltpu.VMEM((1, _LANES), jnp.float32),
            pltpu.VMEM((1, _LANES), jnp.float32),
        ],
        compiler_params=pltpu.CompilerParams(
            dimension_semantics=("arbitrary",),
        ),
    )(x, W_proj, b_proj2, wr_pad, br_pad)


def _dispatch_body(h_hbm, p1_hbm, p2_hbm, hs_hbm,
                   posa_v, posb_v, rows_v, si0, si1, so):
    c = lax.axis_index("c")
    s = lax.axis_index("s")
    wid = s * _NC + c
    base = wid * _TPW
    sls = [pl.ds(base + j * _CH, _CH) for j in range(_NCHUNK)]
    for j in range(_NCHUNK):
        pltpu.sync_copy(p1_hbm.at[sls[j]], posa_v.at[j])
        pltpu.sync_copy(p2_hbm.at[sls[j]], posb_v.at[j])
    si = [si0, si1]
    cin = [None] * _NCHUNK
    cout = [None] * (2 * _NCHUNK)
    cin[0] = pltpu.async_copy(h_hbm.at[sls[0]], rows_v.at[0], si[0])
    for j in range(_NCHUNK):
        if j >= 1 and j + 1 < _NCHUNK:
            cout[2 * (j - 1)].wait()
            cout[2 * (j - 1) + 1].wait()
        if j + 1 < _NCHUNK:
            cin[j + 1] = pltpu.async_copy(
                h_hbm.at[sls[j + 1]], rows_v.at[(j + 1) % 2], si[(j + 1) % 2])
        cin[j].wait()
        cout[2 * j] = pltpu.async_copy(
            rows_v.at[j % 2], hs_hbm.at[posa_v.at[j]], so)
        cout[2 * j + 1] = pltpu.async_copy(
            rows_v.at[j % 2], hs_hbm.at[posb_v.at[j]], so)
    for k in range(2 * (_NCHUNK - 2), 2 * _NCHUNK):
        cout[k].wait()


def _dispatch_call(h, pos1c, pos2c):
    mesh = plsc.VectorSubcoreMesh(core_axis_name="c", subcore_axis_name="s")
    f = functools.partial(
        pl.kernel, mesh=mesh,
        out_type=[
            jax.ShapeDtypeStruct((_ROWS, _H), jnp.float32),
        ],
        scratch_types=[
            pltpu.VMEM((_NCHUNK, _CH), jnp.int32),
            pltpu.VMEM((_NCHUNK, _CH), jnp.int32),
            pltpu.VMEM((2, _CH, _H), jnp.float32),
            pltpu.SemaphoreType.DMA,
            pltpu.SemaphoreType.DMA,
            pltpu.SemaphoreType.DMA,
        ],
    )(_dispatch_body)
    return f(h, pos1c, pos2c)


_KMAX = _CAP // _BE


def _expert_body(tbl_ref, hs_ref, w1_ref, b1_ref, w2_ref, b2_ref, ys_ref):
    e = pl.program_id(0)
    k = pl.program_id(1)

    @pl.when(k < tbl_ref[e])
    def _go():
        hb = hs_ref[...]
        a = jax.nn.gelu(jnp.dot(hb, w1_ref[0], preferred_element_type=jnp.float32)
                        + b1_ref[0])
        ys_ref[...] = jnp.dot(a, w2_ref[0], preferred_element_type=jnp.float32) + b2_ref[0]


def _expert_call(tbl, hs, W1, b1r, W2, b2r):
    grid_spec = pltpu.PrefetchScalarGridSpec(
        num_scalar_prefetch=1,
        grid=(_E, _KMAX),
        in_specs=[
            pl.BlockSpec((_BE, _H), lambda e, k, b: (e * _KMAX + k, 0)),
            pl.BlockSpec((1, _H, _F), lambda e, k, b: (e, 0, 0)),
            pl.BlockSpec((1, 1, _F), lambda e, k, b: (e, 0, 0)),
            pl.BlockSpec((1, _F, _N), lambda e, k, b: (e, 0, 0)),
            pl.BlockSpec((1, 1, _N), lambda e, k, b: (e, 0, 0)),
        ],
        out_specs=pl.BlockSpec((_BE, _N), lambda e, k, b: (e * _KMAX + k, 0)),
    )
    return pl.pallas_call(
        _expert_body,
        grid_spec=grid_spec,
        out_shape=jax.ShapeDtypeStruct((_ROWS, _N), jnp.float32),
        compiler_params=pltpu.CompilerParams(
            dimension_semantics=("arbitrary", "arbitrary"),
        ),
    )(tbl, hs, W1, b1r, W2, b2r)


def _gather_body(ys_hbm, p1_hbm, p2_hbm, y1_hbm, y2_hbm,
                 idx1_v, idx2_v, rows_v, sg0, sg1, sw):
    c = lax.axis_index("c")
    s = lax.axis_index("s")
    wid = s * _NC + c
    base = wid * _TPW
    sls = [pl.ds(base + j * _CH, _CH) for j in range(_NCHUNK)]
    for j in range(_NCHUNK):
        pltpu.sync_copy(p1_hbm.at[sls[j]], idx1_v.at[j])
        pltpu.sync_copy(p2_hbm.at[sls[j]], idx2_v.at[j])

    nops = 2 * _NCHUNK

    def idxref(k):
        return (idx1_v if k % 2 == 0 else idx2_v).at[k // 2]

    def dst(k):
        return (y1_hbm if k % 2 == 0 else y2_hbm).at[sls[k // 2]]

    sg = [sg0, sg1]
    gin = [None] * nops
    wout = [None] * nops
    gin[0] = pltpu.async_copy(ys_hbm.at[idxref(0)], rows_v.at[0], sg[0])
    gin[1] = pltpu.async_copy(ys_hbm.at[idxref(1)], rows_v.at[1], sg[1])
    for k in range(nops):
        gin[k].wait()
        wout[k] = pltpu.async_copy(rows_v.at[k % 2], dst(k), sw)
        if k + 2 < nops:
            wout[k].wait()
            gin[k + 2] = pltpu.async_copy(
                ys_hbm.at[idxref(k + 2)], rows_v.at[k % 2], sg[k % 2])
    wout[nops - 2].wait()
    wout[nops - 1].wait()


def _gather_call(ys, pos1, pos2):
    mesh = plsc.VectorSubcoreMesh(core_axis_name="c", subcore_axis_name="s")
    f = functools.partial(
        pl.kernel, mesh=mesh,
        out_type=[
            jax.ShapeDtypeStruct((_T, _N), jnp.float32),
            jax.ShapeDtypeStruct((_T, _N), jnp.float32),
        ],
        scratch_types=[
            pltpu.VMEM((_NCHUNK, _CH), jnp.int32),
            pltpu.VMEM((_NCHUNK, _CH), jnp.int32),
            pltpu.VMEM((2, _CH, _N), jnp.float32),
            pltpu.SemaphoreType.DMA,
            pltpu.SemaphoreType.DMA,
            pltpu.SemaphoreType.DMA,
        ],
    )(_gather_body)
    return f(ys, pos1, pos2)


def _combine_body(g1_ref, g2_ref, y1_ref, y2_ref, out_ref):
    out_ref[...] = g1_ref[:, :1] * y1_ref[...] + g2_ref[:, :1] * y2_ref[...]


def _combine_call(g1, g2, y1, y2):
    blk = 512
    tok = lambda w: pl.BlockSpec((blk, w), lambda i: (i, 0))
    return pl.pallas_call(
        _combine_body,
        grid=(_T // blk,),
        in_specs=[tok(_LANES), tok(_LANES), tok(_N), tok(_N)],
        out_specs=tok(_N),
        out_shape=jax.ShapeDtypeStruct((_T, _N), jnp.float32),
    )(g1, g2, y1, y2)


def kernel(x, W_proj, b_proj, W_router, b_router, W1, b1, W2, b2):
    wr_pad = jnp.zeros((_H, _LANES), jnp.float32).at[:, :_E].set(W_router)
    br_pad = jnp.full((1, _LANES), -1e30, jnp.float32).at[0, :_E].set(b_router)
    b_proj2 = b_proj.reshape(1, _H)
    b1r = b1[:, None, :]
    b2r = b2[:, None, :]

    (h, g1, g2, pos1, pos2, counts, aux) = _router_call(
        x, W_proj, b_proj2, wr_pad, br_pad)

    cnt = counts[0, :_E].astype(jnp.int32)
    tbl = (cnt + (_BE - 1)) // _BE

    pos1c = pos1[:, 0]
    pos2c = pos2[:, 0]

    (hs,) = _dispatch_call(h, pos1c, pos2c)
    ys = _expert_call(tbl, hs, W1, b1r, W2, b2r)
    y1, y2 = _gather_call(ys, pos1c, pos2c)
    out = _combine_call(g1, g2, y1, y2)
    return out, aux[0, 0]


# speedup vs baseline: 1.4124x; 1.4124x over previous
import functools

import jax
import jax.numpy as jnp
from jax import lax
from jax.experimental import pallas as pl
from jax.experimental.pallas import tpu as pltpu
from jax.experimental.pallas import tpu_sc as plsc

_T = 2048
_D = 1024
_H = 1024
_N = 1024
_E = 8
_F = 2048
_BT = 256
_LANES = 128
_BE = 256
_CAP = 2048
_BEX = 512
_KMAXX = _CAP // _BEX
_NBCX = _E * _KMAXX
_NBX = 15
_ROWS = _NBCX * _BEX + _BEX

_NC = 2
_NS = 16
_NW = _NC * _NS
_TPW = _T // _NW
_CH = 16
_NCHUNK = _TPW // _CH


def _router_body(x_ref, wp_ref, bp_ref, wr_ref, br_ref,
                 h_ref, g1_ref, g2_ref, p1_ref, p2_ref,
                 counts_ref, aux_ref, cnt_acc, imp_acc):
    pid = pl.program_id(0)
    nblk = pl.num_programs(0)

    @pl.when(pid == 0)
    def _init():
        cnt_acc[...] = jnp.zeros_like(cnt_acc)
        imp_acc[...] = jnp.zeros_like(imp_acc)

    x = x_ref[...]
    h = jax.nn.gelu(jnp.dot(x, wp_ref[...], preferred_element_type=jnp.float32)
                    + bp_ref[...])
    h_ref[...] = h

    logits = jnp.dot(h, wr_ref[...], preferred_element_type=jnp.float32) + br_ref[...]
    lmax = jnp.max(logits, axis=-1, keepdims=True)
    ex = jnp.exp(logits - lmax)
    probs = ex / jnp.sum(ex, axis=-1, keepdims=True)

    lane = jax.lax.broadcasted_iota(jnp.int32, probs.shape, 1)
    v1 = jnp.max(probs, axis=-1, keepdims=True)
    i1 = jnp.min(jnp.where(probs >= v1, lane, _LANES), axis=-1, keepdims=True)
    m1 = lane == i1
    probs_m = jnp.where(m1, -1.0, probs)
    v2 = jnp.max(probs_m, axis=-1, keepdims=True)
    i2 = jnp.min(jnp.where(probs_m >= v2, lane, _LANES), axis=-1, keepdims=True)
    m2 = lane == i2

    s = v1 + v2
    g1 = v1 / s
    g2 = v2 / s

    mask = (m1 | m2).astype(jnp.float32)
    row = jax.lax.broadcasted_iota(jnp.int32, (_BT, _BT), 0)
    col = jax.lax.broadcasted_iota(jnp.int32, (_BT, _BT), 1)
    ltri = (col < row).astype(jnp.float32)
    excl = jnp.dot(ltri, mask, preferred_element_type=jnp.float32)
    carry = cnt_acc[...]
    rank = excl + carry
    r1 = jnp.sum(jnp.where(m1, rank, 0.0), axis=-1, keepdims=True)
    r2 = jnp.sum(jnp.where(m2, rank, 0.0), axis=-1, keepdims=True)

    pos1 = i1.astype(jnp.float32) * _CAP + r1
    pos2 = i2.astype(jnp.float32) * _CAP + r2

    ones = jnp.ones((1, _LANES), jnp.float32)
    g1_ref[...] = g1 * ones
    g2_ref[...] = g2 * ones
    p1_ref[...] = (pos1 * ones).astype(jnp.int32)
    p2_ref[...] = (pos2 * ones).astype(jnp.int32)

    cnt_new = carry + jnp.sum(mask, axis=0, keepdims=True)
    imp_new = imp_acc[...] + jnp.sum(probs, axis=0, keepdims=True)
    cnt_acc[...] = cnt_new
    imp_acc[...] = imp_new

    @pl.when(pid == nblk - 1)
    def _fin():
        counts_ref[...] = cnt_new
        aux = _E * jnp.sum((imp_new / _T) * (cnt_new / _T))
        aux_ref[...] = jnp.full((1, _LANES), aux, jnp.float32)


def _router_call(x, W_proj, b_proj2, wr_pad, br_pad):
    nblk = _T // _BT
    out_shapes = (
        jax.ShapeDtypeStruct((_T, _H), jnp.float32),
        jax.ShapeDtypeStruct((_T, _LANES), jnp.float32),
        jax.ShapeDtypeStruct((_T, _LANES), jnp.float32),
        jax.ShapeDtypeStruct((_T, _LANES), jnp.int32),
        jax.ShapeDtypeStruct((_T, _LANES), jnp.int32),
        jax.ShapeDtypeStruct((1, _LANES), jnp.float32),
        jax.ShapeDtypeStruct((1, _LANES), jnp.float32),
    )
    tok_spec = lambda w: pl.BlockSpec((_BT, w), lambda i: (i, 0))
    const_spec = lambda a, b: pl.BlockSpec((a, b), lambda i: (0, 0))
    return pl.pallas_call(
        _router_body,
        grid=(nblk,),
        in_specs=[
            tok_spec(_D),
            const_spec(_D, _H),
            const_spec(1, _H),
            const_spec(_H, _LANES),
            const_spec(1, _LANES),
        ],
        out_specs=(
            tok_spec(_H),
            tok_spec(_LANES), tok_spec(_LANES),
            tok_spec(_LANES), tok_spec(_LANES),
            const_spec(1, _LANES), const_spec(1, _LANES),
        ),
        out_shape=out_shapes,
        scratch_shapes=[
            pltpu.VMEM((1, _LANES), jnp.float32),
            pltpu.VMEM((1, _LANES), jnp.float32),
        ],
        compiler_params=pltpu.CompilerParams(
            dimension_semantics=("arbitrary",),
        ),
    )(x, W_proj, b_proj2, wr_pad, br_pad)


def _dispatch_body(h_hbm, p1_hbm, p2_hbm, hs_hbm,
                   posa_v, posb_v, rows_v, si0, si1, so):
    c = lax.axis_index("c")
    s = lax.axis_index("s")
    wid = s * _NC + c
    base = wid * _TPW
    sls = [pl.ds(base + j * _CH, _CH) for j in range(_NCHUNK)]
    for j in range(_NCHUNK):
        pltpu.sync_copy(p1_hbm.at[sls[j]], posa_v.at[j])
        pltpu.sync_copy(p2_hbm.at[sls[j]], posb_v.at[j])
    si = [si0, si1]
    cin = [None] * _NCHUNK
    cout = [None] * (2 * _NCHUNK)
    cin[0] = pltpu.async_copy(h_hbm.at[sls[0]], rows_v.at[0], si[0])
    for j in range(_NCHUNK):
        if j >= 1 and j + 1 < _NCHUNK:
            cout[2 * (j - 1)].wait()
            cout[2 * (j - 1) + 1].wait()
        if j + 1 < _NCHUNK:
            cin[j + 1] = pltpu.async_copy(
                h_hbm.at[sls[j + 1]], rows_v.at[(j + 1) % 2], si[(j + 1) % 2])
        cin[j].wait()
        cout[2 * j] = pltpu.async_copy(
            rows_v.at[j % 2], hs_hbm.at[posa_v.at[j]], so)
        cout[2 * j + 1] = pltpu.async_copy(
            rows_v.at[j % 2], hs_hbm.at[posb_v.at[j]], so)
    for k in range(2 * (_NCHUNK - 2), 2 * _NCHUNK):
        cout[k].wait()


def _dispatch_call(h, pos1c, pos2c):
    mesh = plsc.VectorSubcoreMesh(core_axis_name="c", subcore_axis_name="s")
    f = functools.partial(
        pl.kernel, mesh=mesh,
        out_type=[
            jax.ShapeDtypeStruct((_ROWS, _H), jnp.float32),
        ],
        scratch_types=[
            pltpu.VMEM((_NCHUNK, _CH), jnp.int32),
            pltpu.VMEM((_NCHUNK, _CH), jnp.int32),
            pltpu.VMEM((2, _CH, _H), jnp.float32),
            pltpu.SemaphoreType.DMA,
            pltpu.SemaphoreType.DMA,
            pltpu.SemaphoreType.DMA,
        ],
    )(_dispatch_body)
    return f(h, pos1c, pos2c)


def _expert_body(tbl_ref, hs_ref, w1_ref, b1_ref, w2_ref, b2_ref, ys_ref):
    i = pl.program_id(0)
    nb = tbl_ref[2 * _NBX]

    @pl.when(i < nb)
    def _go():
        hb = hs_ref[...]
        a = jax.nn.gelu(jnp.dot(hb, w1_ref[0], preferred_element_type=jnp.float32)
                        + b1_ref[0])
        ys_ref[...] = jnp.dot(a, w2_ref[0], preferred_element_type=jnp.float32) + b2_ref[0]


def _expert_call(tbl, hs, W1, b1r, W2, b2r):
    grid_spec = pltpu.PrefetchScalarGridSpec(
        num_scalar_prefetch=1,
        grid=(_NBX,),
        in_specs=[
            pl.BlockSpec((_BEX, _H), lambda i, b: (b[i], 0)),
            pl.BlockSpec((1, _H, _F), lambda i, b: (b[_NBX + i], 0, 0)),
            pl.BlockSpec((1, 1, _F), lambda i, b: (b[_NBX + i], 0, 0)),
            pl.BlockSpec((1, _F, _N), lambda i, b: (b[_NBX + i], 0, 0)),
            pl.BlockSpec((1, 1, _N), lambda i, b: (b[_NBX + i], 0, 0)),
        ],
        out_specs=pl.BlockSpec((_BEX, _N), lambda i, b: (b[i], 0)),
    )
    return pl.pallas_call(
        _expert_body,
        grid_spec=grid_spec,
        out_shape=jax.ShapeDtypeStruct((_ROWS, _N), jnp.float32),
        compiler_params=pltpu.CompilerParams(
            dimension_semantics=("arbitrary",),
        ),
    )(tbl, hs, W1, b1r, W2, b2r)


def _gather_body(ys_hbm, p1_hbm, p2_hbm, y1_hbm, y2_hbm,
                 idx1_v, idx2_v, rows_v, sg0, sg1, sw):
    c = lax.axis_index("c")
    s = lax.axis_index("s")
    wid = s * _NC + c
    base = wid * _TPW
    sls = [pl.ds(base + j * _CH, _CH) for j in range(_NCHUNK)]
    for j in range(_NCHUNK):
        pltpu.sync_copy(p1_hbm.at[sls[j]], idx1_v.at[j])
        pltpu.sync_copy(p2_hbm.at[sls[j]], idx2_v.at[j])

    nops = 2 * _NCHUNK

    def idxref(k):
        return (idx1_v if k % 2 == 0 else idx2_v).at[k // 2]

    def dst(k):
        return (y1_hbm if k % 2 == 0 else y2_hbm).at[sls[k // 2]]

    sg = [sg0, sg1]
    gin = [None] * nops
    wout = [None] * nops
    gin[0] = pltpu.async_copy(ys_hbm.at[idxref(0)], rows_v.at[0], sg[0])
    gin[1] = pltpu.async_copy(ys_hbm.at[idxref(1)], rows_v.at[1], sg[1])
    for k in range(nops):
        gin[k].wait()
        wout[k] = pltpu.async_copy(rows_v.at[k % 2], dst(k), sw)
        if k + 2 < nops:
            wout[k].wait()
            gin[k + 2] = pltpu.async_copy(
                ys_hbm.at[idxref(k + 2)], rows_v.at[k % 2], sg[k % 2])
    wout[nops - 2].wait()
    wout[nops - 1].wait()


def _gather_call(ys, pos1, pos2):
    mesh = plsc.VectorSubcoreMesh(core_axis_name="c", subcore_axis_name="s")
    f = functools.partial(
        pl.kernel, mesh=mesh,
        out_type=[
            jax.ShapeDtypeStruct((_T, _N), jnp.float32),
            jax.ShapeDtypeStruct((_T, _N), jnp.float32),
        ],
        scratch_types=[
            pltpu.VMEM((_NCHUNK, _CH), jnp.int32),
            pltpu.VMEM((_NCHUNK, _CH), jnp.int32),
            pltpu.VMEM((2, _CH, _N), jnp.float32),
            pltpu.SemaphoreType.DMA,
            pltpu.SemaphoreType.DMA,
            pltpu.SemaphoreType.DMA,
        ],
    )(_gather_body)
    return f(ys, pos1, pos2)


def _combine_body(g1_ref, g2_ref, y1_ref, y2_ref, out_ref):
    out_ref[...] = g1_ref[:, :1] * y1_ref[...] + g2_ref[:, :1] * y2_ref[...]


def _combine_call(g1, g2, y1, y2):
    blk = 512
    tok = lambda w: pl.BlockSpec((blk, w), lambda i: (i, 0))
    return pl.pallas_call(
        _combine_body,
        grid=(_T // blk,),
        in_specs=[tok(_LANES), tok(_LANES), tok(_N), tok(_N)],
        out_specs=tok(_N),
        out_shape=jax.ShapeDtypeStruct((_T, _N), jnp.float32),
    )(g1, g2, y1, y2)


def kernel(x, W_proj, b_proj, W_router, b_router, W1, b1, W2, b2):
    wr_pad = jnp.zeros((_H, _LANES), jnp.float32).at[:, :_E].set(W_router)
    br_pad = jnp.full((1, _LANES), -1e30, jnp.float32).at[0, :_E].set(b_router)
    b_proj2 = b_proj.reshape(1, _H)
    b1r = b1[:, None, :]
    b2r = b2[:, None, :]

    (h, g1, g2, pos1, pos2, counts, aux) = _router_call(
        x, W_proj, b_proj2, wr_pad, br_pad)

    cnt = counts[0, :_E].astype(jnp.int32)
    nblk_e = (cnt + (_BEX - 1)) // _BEX
    cumnb = jnp.cumsum(nblk_e)
    nb_total = cumnb[_E - 1]
    i_ar = jnp.arange(_NBX, dtype=jnp.int32)
    e_i = jnp.searchsorted(cumnb, i_ar, side="right").astype(jnp.int32)
    e_c = jnp.minimum(e_i, _E - 1)
    start = cumnb[e_c] - nblk_e[e_c]
    bidx = e_c * _KMAXX + (i_ar - start)
    valid = i_ar < nb_total
    bidx = jnp.where(valid, bidx, _NBCX).astype(jnp.int32)
    eidx = jnp.where(valid, e_c, _E - 1).astype(jnp.int32)
    tbl = jnp.concatenate([bidx, eidx, nb_total[None].astype(jnp.int32)])

    pos1c = pos1[:, 0]
    pos2c = pos2[:, 0]

    (hs,) = _dispatch_call(h, pos1c, pos2c)
    ys = _expert_call(tbl, hs, W1, b1r, W2, b2r)
    y1, y2 = _gather_call(ys, pos1c, pos2c)
    out = _combine_call(g1, g2, y1, y2)
    return out, aux[0, 0]
